# Initial kernel scaffold; baseline (speedup 1.0000x reference)
import functools

import jax
import jax.numpy as jnp
from jax import lax
from jax.experimental import pallas as pl
from jax.experimental.pallas import tpu as pltpu
from jax.experimental.pallas import tpu_sc as plsc

N = 10000
E = 320000
D_IN = 128
D_HID = 128
D_OUT = 40
D_PAD = 64

NC = 2
NS = 16
NW = NC * NS
EPW = E // NW
CH = 80
NCHUNK = EPW // CH

ZR = 640
ZR_LAST = N - 15 * ZR


def _make_scatter(D):
    mesh = plsc.VectorSubcoreMesh(core_axis_name="c", subcore_axis_name="s")

    @functools.partial(
        pl.kernel,
        out_type=jax.ShapeDtypeStruct((NC, N, D), jnp.float32),
        mesh=mesh,
        scratch_types=[
            pltpu.VMEM((CH,), jnp.int32),
            pltpu.VMEM((CH,), jnp.int32),
            pltpu.VMEM((CH, D), jnp.float32),
            pltpu.VMEM_SHARED((N, D), jnp.float32),
            pltpu.SemaphoreType.DMA,
        ],
    )
    def scatter_kernel(h_hbm, src_hbm, dst_hbm, zero_hbm, out_hbm,
                       src_v, dst_v, rows_v, acc_sh, sem):
        c = lax.axis_index("c")
        s = lax.axis_index("s")
        wid = s * NC + c

        @pl.when(s < 15)
        def _():
            pltpu.sync_copy(zero_hbm.at[pl.ds(s * ZR, ZR)],
                            acc_sh.at[pl.ds(s * ZR, ZR)])

        @pl.when(s == 15)
        def _():
            pltpu.sync_copy(zero_hbm.at[pl.ds(15 * ZR, ZR_LAST)],
                            acc_sh.at[pl.ds(15 * ZR, ZR_LAST)])

        plsc.subcore_barrier()

        base0 = wid * EPW

        @pl.loop(0, NCHUNK)
        def _(i):
            base = base0 + i * CH
            pltpu.sync_copy(src_hbm.at[pl.ds(base, CH)], src_v)
            pltpu.sync_copy(dst_hbm.at[pl.ds(base, CH)], dst_v)
            pltpu.async_copy(h_hbm.at[src_v], rows_v, sem).wait()
            pltpu.sync_copy(rows_v, acc_sh.at[dst_v], add=True)

        plsc.subcore_barrier()

        @pl.when(s < 15)
        def _():
            pltpu.sync_copy(acc_sh.at[pl.ds(s * ZR, ZR)],
                            out_hbm.at[c].at[pl.ds(s * ZR, ZR)])

        @pl.when(s == 15)
        def _():
            pltpu.sync_copy(acc_sh.at[pl.ds(15 * ZR, ZR_LAST)],
                            out_hbm.at[c].at[pl.ds(15 * ZR, ZR_LAST)])

    return scatter_kernel


_scatter128 = _make_scatter(D_HID)
_scatter64 = _make_scatter(D_PAD)


def _mm(x, w):
    def body(x_ref, w_ref, o_ref):
        o_ref[...] = jnp.dot(x_ref[...], w_ref[...],
                             preferred_element_type=jnp.float32)

    return pl.pallas_call(
        body,
        out_shape=jax.ShapeDtypeStruct((x.shape[0], w.shape[1]), jnp.float32),
    )(x, w)


def _fuse_mm(p, b, w):
    def body(p_ref, b_ref, w_ref, o_ref):
        h = jnp.maximum(p_ref[0] + p_ref[1] + b_ref[...], 0.0)
        o_ref[...] = jnp.dot(h, w_ref[...], preferred_element_type=jnp.float32)

    return pl.pallas_call(
        body,
        out_shape=jax.ShapeDtypeStruct((p.shape[1], w.shape[1]), jnp.float32),
    )(p, b.reshape(1, -1), w)


def _finalize(q, b2p):
    def body(q_ref, b_ref, o_ref):
        h = q_ref[0] + q_ref[1] + b_ref[...]
        m = jnp.max(h, axis=1, keepdims=True)
        e = jnp.exp(h - m)
        lse = jnp.log(jnp.sum(e, axis=1, keepdims=True))
        o_ref[...] = h - m - lse

    return pl.pallas_call(
        body,
        out_shape=jax.ShapeDtypeStruct((q.shape[1], q.shape[2]), jnp.float32),
    )(q, b2p.reshape(1, -1))


def kernel(x, edge_index, W0, b0, W1, b1, W2, b2):
    src = edge_index[0]
    dst = edge_index[1]
    zeros128 = jnp.zeros((N, D_HID), jnp.float32)
    zeros64 = jnp.zeros((N, D_PAD), jnp.float32)
    W2p = jnp.pad(W2, ((0, 0), (0, D_PAD - D_OUT)))
    b2p = jnp.concatenate(
        [b2, jnp.full((D_PAD - D_OUT,), -1e30, jnp.float32)])

    h = _mm(x, W0)
    p = _scatter128(h, src, dst, zeros128)
    h = _fuse_mm(p, b0, W1)
    p = _scatter128(h, src, dst, zeros128)
    h = _fuse_mm(p, b1, W2p)
    q = _scatter64(h, src, dst, zeros64)
    out = _finalize(q, b2p)
    return out[:, :D_OUT]


# speedup vs baseline: 5.0050x; 5.0050x over previous
import functools

import jax
import jax.numpy as jnp
from jax import lax
from jax.experimental import pallas as pl
from jax.experimental.pallas import tpu as pltpu
from jax.experimental.pallas import tpu_sc as plsc

N = 10000
E = 320000
D_IN = 128
D_HID = 128
D_OUT = 40
D_PAD = 128

NC = 2
NS = 16
NW = NC * NS
EPW = E // NW
CH = 80
NCHUNK = EPW // CH

ZR = 640
ZR_LAST = N - 15 * ZR


def _make_scatter(D):
    mesh = plsc.VectorSubcoreMesh(core_axis_name="c", subcore_axis_name="s")

    @functools.partial(
        pl.kernel,
        out_type=jax.ShapeDtypeStruct((NC, N, D), jnp.float32),
        mesh=mesh,
        scratch_types=[
            pltpu.VMEM((CH,), jnp.int32),
            pltpu.VMEM((CH,), jnp.int32),
            pltpu.VMEM((CH, D), jnp.float32),
            pltpu.VMEM_SHARED((N, D), jnp.float32),
            pltpu.SemaphoreType.DMA,
        ],
    )
    def scatter_kernel(h_hbm, src_hbm, dst_hbm, zero_hbm, out_hbm,
                       src_v, dst_v, rows_v, acc_sh, sem):
        c = lax.axis_index("c")
        s = lax.axis_index("s")
        wid = s * NC + c

        @pl.when(s < 15)
        def _():
            pltpu.sync_copy(zero_hbm.at[pl.ds(s * ZR, ZR)],
                            acc_sh.at[pl.ds(s * ZR, ZR)])

        @pl.when(s == 15)
        def _():
            pltpu.sync_copy(zero_hbm.at[pl.ds(15 * ZR, ZR_LAST)],
                            acc_sh.at[pl.ds(15 * ZR, ZR_LAST)])

        plsc.subcore_barrier()

        base0 = wid * EPW

        @pl.loop(0, NCHUNK)
        def _(i):
            base = base0 + i * CH
            pltpu.sync_copy(src_hbm.at[pl.ds(base, CH)], src_v)
            pltpu.sync_copy(dst_hbm.at[pl.ds(base, CH)], dst_v)
            pltpu.async_copy(h_hbm.at[src_v], rows_v, sem).wait()
            pltpu.sync_copy(rows_v, acc_sh.at[dst_v], add=True)

        plsc.subcore_barrier()

        @pl.when(s < 15)
        def _():
            pltpu.sync_copy(acc_sh.at[pl.ds(s * ZR, ZR)],
                            out_hbm.at[c].at[pl.ds(s * ZR, ZR)])

        @pl.when(s == 15)
        def _():
            pltpu.sync_copy(acc_sh.at[pl.ds(15 * ZR, ZR_LAST)],
                            out_hbm.at[c].at[pl.ds(15 * ZR, ZR_LAST)])

    return scatter_kernel


_scatter128 = _make_scatter(D_HID)


def _mm(x, w):
    def body(x_ref, w_ref, o_ref):
        o_ref[...] = jnp.dot(x_ref[...], w_ref[...],
                             preferred_element_type=jnp.float32)

    return pl.pallas_call(
        body,
        out_shape=jax.ShapeDtypeStruct((x.shape[0], w.shape[1]), jnp.float32),
    )(x, w)


def _fuse_mm(p, b, w):
    def body(p_ref, b_ref, w_ref, o_ref):
        h = jnp.maximum(p_ref[0] + p_ref[1] + b_ref[...], 0.0)
        o_ref[...] = jnp.dot(h, w_ref[...], preferred_element_type=jnp.float32)

    return pl.pallas_call(
        body,
        out_shape=jax.ShapeDtypeStruct((p.shape[1], w.shape[1]), jnp.float32),
    )(p, b.reshape(1, -1), w)


def _finalize(q, b2p):
    def body(q_ref, b_ref, o_ref):
        h = q_ref[0] + q_ref[1] + b_ref[...]
        m = jnp.max(h, axis=1, keepdims=True)
        e = jnp.exp(h - m)
        lse = jnp.log(jnp.sum(e, axis=1, keepdims=True))
        o_ref[...] = h - m - lse

    return pl.pallas_call(
        body,
        out_shape=jax.ShapeDtypeStruct((q.shape[1], q.shape[2]), jnp.float32),
    )(q, b2p.reshape(1, -1))


def kernel(x, edge_index, W0, b0, W1, b1, W2, b2):
    src = edge_index[0]
    dst = edge_index[1]
    zeros128 = jnp.zeros((N, D_HID), jnp.float32)
    W2p = jnp.pad(W2, ((0, 0), (0, D_PAD - D_OUT)))
    b2p = jnp.concatenate(
        [b2, jnp.full((D_PAD - D_OUT,), -1e30, jnp.float32)])

    h = _mm(x, W0)
    p = _scatter128(h, src, dst, zeros128)
    h = _fuse_mm(p, b0, W1)
    p = _scatter128(h, src, dst, zeros128)
    h = _fuse_mm(p, b1, W2p)
    q = _scatter128(h, src, dst, zeros128)
    out = _finalize(q, b2p)
    return out[:, :D_OUT]
